# baseline (speedup 1.0000x reference)
import jax
import jax.numpy as jnp
from jax.experimental import pallas as pl
from jax.experimental.pallas import tpu as pltpu

_OUT = 3
_K = 256
_N_PAD = 128


def _fc3t_kernel(x_ref, w_ref, b_ref, o_ref):
    x = jnp.maximum(x_ref[...], 0.0).astype(jnp.bfloat16)
    acc = jax.lax.dot_general(
        w_ref[...].astype(jnp.bfloat16), x, (((0,), (0,)), ((), ())),
        preferred_element_type=jnp.float32)
    b_col = b_ref[...].reshape(_N_PAD, 1)
    o_ref[...] = (acc + b_col)[:_OUT, :]


def kernel(x_nchw, w_pad, b_pad, *, tn=16384):
    B = x_nchw.shape[0]
    x_t = x_nchw.reshape(B, _K).T

    Bp = pl.cdiv(B, tn) * tn
    if Bp != B:
        x_t = jnp.pad(x_t, ((0, 0), (0, Bp - B)))

    out_t = pl.pallas_call(
        _fc3t_kernel,
        out_shape=jax.ShapeDtypeStruct((_OUT, Bp), jnp.float32),
        grid=(Bp // tn,),
        in_specs=[
            pl.BlockSpec((_K, tn), lambda i: (0, i)),
            pl.BlockSpec((_K, _N_PAD), lambda i: (0, 0)),
            pl.BlockSpec((1, _N_PAD), lambda i: (0, 0)),
        ],
        out_specs=pl.BlockSpec((_OUT, tn), lambda i: (0, i)),
        compiler_params=pltpu.CompilerParams(
            dimension_semantics=("parallel",)),
    )(x_t, w_pad, b_pad)

    return out_t[:, :B].T


# speedup vs baseline: 1.0024x; 1.0024x over previous
import jax
import jax.numpy as jnp
from jax.experimental import pallas as pl
from jax.experimental.pallas import tpu as pltpu

_OUT = 3
_K = 256
_N_PAD = 128


def _fc3t_kernel(x_ref, w_ref, b_ref, o_ref):
    x = jnp.maximum(x_ref[...], 0.0)
    acc = jax.lax.dot_general(
        w_ref[...], x, (((0,), (0,)), ((), ())),
        preferred_element_type=jnp.float32)
    b_col = b_ref[...].reshape(_N_PAD, 1)
    o_ref[...] = acc[:_OUT, :] + b_col[:_OUT, :]


def kernel(x_nchw, w_pad, b_pad, *, tn=16384):
    B = x_nchw.shape[0]
    x_t = x_nchw.reshape(B, _K).T

    Bp = pl.cdiv(B, tn) * tn
    if Bp != B:
        x_t = jnp.pad(x_t, ((0, 0), (0, Bp - B)))

    out_t = pl.pallas_call(
        _fc3t_kernel,
        out_shape=jax.ShapeDtypeStruct((_OUT, Bp), jnp.float32),
        grid=(Bp // tn,),
        in_specs=[
            pl.BlockSpec((_K, tn), lambda i: (0, i)),
            pl.BlockSpec((_K, _N_PAD), lambda i: (0, 0)),
            pl.BlockSpec((1, _N_PAD), lambda i: (0, 0)),
        ],
        out_specs=pl.BlockSpec((_OUT, tn), lambda i: (0, i)),
        compiler_params=pltpu.CompilerParams(
            dimension_semantics=("parallel",)),
    )(x_t, w_pad, b_pad)

    return out_t[:, :B].T


# speedup vs baseline: 1.0034x; 1.0010x over previous
import jax
import jax.numpy as jnp
from jax.experimental import pallas as pl
from jax.experimental.pallas import tpu as pltpu

_OUT = 3
_K = 256
_N_PAD = 128


def _fc3t_kernel(x_ref, w_ref, b_ref, o_ref):
    x = jnp.maximum(x_ref[...], 0.0)
    acc = jax.lax.dot_general(
        w_ref[...], x, (((0,), (0,)), ((), ())),
        preferred_element_type=jnp.float32)
    b_col = b_ref[...].reshape(_N_PAD, 1)
    o_ref[...] = (acc + b_col)[:_OUT, :]


def kernel(x_nchw, w_pad, b_pad, *, tn=16384):
    B = x_nchw.shape[0]
    x_t = x_nchw.reshape(B, _K).T

    Bp = pl.cdiv(B, tn) * tn
    if Bp != B:
        x_t = jnp.pad(x_t, ((0, 0), (0, Bp - B)))

    out_t = pl.pallas_call(
        _fc3t_kernel,
        out_shape=jax.ShapeDtypeStruct((_OUT, Bp), jnp.float32),
        grid=(Bp // tn,),
        in_specs=[
            pl.BlockSpec((_K, tn), lambda i: (0, i)),
            pl.BlockSpec((_K, _N_PAD), lambda i: (0, 0)),
            pl.BlockSpec((1, _N_PAD), lambda i: (0, 0)),
        ],
        out_specs=pl.BlockSpec((_OUT, tn), lambda i: (0, i)),
        compiler_params=pltpu.CompilerParams(
            dimension_semantics=("parallel",)),
    )(x_t, w_pad, b_pad)

    return out_t[:, :B].T


# speedup vs baseline: 1.0336x; 1.0301x over previous
import jax
import jax.numpy as jnp
from jax.experimental import pallas as pl
from jax.experimental.pallas import tpu as pltpu

_OUT = 3
_K = 256
_N_PAD = 128


def _fc3t_kernel(x_ref, w_ref, b_ref, o_ref):
    x = jnp.maximum(x_ref[...], 0.0)
    acc = jax.lax.dot_general(
        w_ref[...], x, (((0,), (0,)), ((), ())),
        preferred_element_type=jnp.float32)
    b_col = b_ref[...].reshape(_N_PAD, 1)
    o_ref[...] = (acc + b_col)[:_OUT, :]


def kernel(x_nchw, w_pad, b_pad, *, tn=8192):
    B = x_nchw.shape[0]
    x_t = x_nchw.reshape(B, _K).T

    Bp = pl.cdiv(B, tn) * tn
    if Bp != B:
        x_t = jnp.pad(x_t, ((0, 0), (0, Bp - B)))

    out_t = pl.pallas_call(
        _fc3t_kernel,
        out_shape=jax.ShapeDtypeStruct((_OUT, Bp), jnp.float32),
        grid=(Bp // tn,),
        in_specs=[
            pl.BlockSpec((_K, tn), lambda i: (0, i)),
            pl.BlockSpec((_K, _N_PAD), lambda i: (0, 0)),
            pl.BlockSpec((1, _N_PAD), lambda i: (0, 0)),
        ],
        out_specs=pl.BlockSpec((_OUT, tn), lambda i: (0, i)),
        compiler_params=pltpu.CompilerParams(
            dimension_semantics=("parallel",)),
    )(x_t, w_pad, b_pad)

    return out_t[:, :B].T
